# baseline (speedup 1.0000x reference)
import jax
import jax.numpy as jnp
from jax.experimental import pallas as pl
from jax.experimental.pallas import tpu as pltpu


_CHUNK_ROWS = (2048, 6144)


def _copy_kernel(src_ref, dst_ref, buf, in_sems, out_sems):
    offs = [0]
    for r in _CHUNK_ROWS[:-1]:
        offs.append(offs[-1] + r)
    ins, outs = [], []
    for i, (o, r) in enumerate(zip(offs, _CHUNK_ROWS)):
        c = pltpu.make_async_copy(
            src_ref.at[pl.ds(o, r)], buf.at[pl.ds(o, r)], in_sems.at[i]
        )
        c.start()
        ins.append(c)
        outs.append(
            pltpu.make_async_copy(
                buf.at[pl.ds(o, r)], dst_ref.at[pl.ds(o, r)], out_sems.at[i]
            )
        )
    for i in range(len(_CHUNK_ROWS)):
        ins[i].wait()
        outs[i].start()
    for c in outs:
        c.wait()


def kernel(prototypes):
    rows, feat = prototypes.shape
    n = len(_CHUNK_ROWS)
    return pl.pallas_call(
        _copy_kernel,
        out_shape=jax.ShapeDtypeStruct(prototypes.shape, prototypes.dtype),
        in_specs=[pl.BlockSpec(memory_space=pl.ANY)],
        out_specs=pl.BlockSpec(memory_space=pl.ANY),
        scratch_shapes=[
            pltpu.VMEM((rows, feat), prototypes.dtype),
            pltpu.SemaphoreType.DMA((n,)),
            pltpu.SemaphoreType.DMA((n,)),
        ],
    )(prototypes)


# speedup vs baseline: 1.1354x; 1.1354x over previous
import jax
import jax.numpy as jnp
from jax.experimental import pallas as pl
from jax.experimental.pallas import tpu as pltpu


_BLOCK_ROWS = 4096


def _copy_kernel(src_ref, dst_ref):
    dst_ref[...] = src_ref[...]


def kernel(prototypes):
    rows, feat = prototypes.shape
    return pl.pallas_call(
        _copy_kernel,
        out_shape=jax.ShapeDtypeStruct(prototypes.shape, prototypes.dtype),
        grid=(rows // _BLOCK_ROWS,),
        in_specs=[pl.BlockSpec((_BLOCK_ROWS, feat), lambda i: (i, 0))],
        out_specs=pl.BlockSpec((_BLOCK_ROWS, feat), lambda i: (i, 0)),
    )(prototypes)
